# baseline (speedup 1.0000x reference)
import jax
import jax.numpy as jnp
from jax import lax
from jax.experimental import pallas as pl
from jax.experimental.pallas import tpu as pltpu
from jax.experimental.pallas import tpu_sc as plsc

_VOCAB = 100000
_EMBED_DIM = 64
_BATCH = 4096
_HIST = 50

_NC = 2
_NS = 16
_NW = _NC * _NS

_BT = _BATCH // _NW
_L = 16


def _transpose_block(buf, buf_t):
  def te_body(te, carry):
    for se in range(8):
      col = jnp.full((_L,), te * 8 + se, jnp.int32)
      for v in range(_BT // _L):
        rows = v * _L + lax.iota(jnp.int32, _L)
        vals = plsc.load_gather(buf, [rows, col])
        buf_t[te, se, pl.ds(v * _L, _L)] = vals
    return carry

  lax.fori_loop(0, 8, te_body, 0)


def _gather_body(table_hbm, idx_hbm, out_hbm, idx_v, buf0, buf1, bt0, bt1,
                 sem_g0, sem_g1, sem_w0, sem_w1):
  wid = lax.axis_index("s") * _NC + lax.axis_index("c")
  pltpu.sync_copy(idx_hbm.at[pl.ds(0, _HIST), pl.ds(wid * _BT, _BT)], idx_v)

  pltpu.async_copy(table_hbm.at[idx_v.at[0]], buf0, sem_g0)

  def step(h, buf, buf_t, sem_g, sem_w, first, last):
    pltpu.make_async_copy(table_hbm.at[idx_v.at[h]], buf, sem_g).wait()

    @pl.when(jnp.logical_not(first))
    def _drain_prev_write():
      pltpu.make_async_copy(buf_t, out_hbm.at[h - 2, pl.ds(0, 8), wid],
                            sem_w).wait()

    _transpose_block(buf, buf_t)
    pltpu.async_copy(buf_t, out_hbm.at[h, pl.ds(0, 8), wid], sem_w)

  def superstep(i, carry):
    h0 = 2 * i
    h1 = 2 * i + 1
    pltpu.async_copy(table_hbm.at[idx_v.at[h1]], buf1, sem_g1)
    step(h0, buf0, bt0, sem_g0, sem_w0, i == 0, False)

    @pl.when(i < _HIST // 2 - 1)
    def _fire_next():
      pltpu.async_copy(table_hbm.at[idx_v.at[h1 + 1]], buf0, sem_g0)

    step(h1, buf1, bt1, sem_g1, sem_w1, i == 0, i == _HIST // 2 - 1)
    return carry

  lax.fori_loop(0, _HIST // 2, superstep, 0)
  pltpu.make_async_copy(bt0, out_hbm.at[_HIST - 2, pl.ds(0, 8), wid],
                        sem_w0).wait()
  pltpu.make_async_copy(bt1, out_hbm.at[_HIST - 1, pl.ds(0, 8), wid],
                        sem_w1).wait()


@jax.jit
def kernel(indices, table):
  idx_t = indices.T.astype(jnp.int32)
  mesh = plsc.VectorSubcoreMesh(core_axis_name="c", subcore_axis_name="s")
  out5 = pl.kernel(
      _gather_body,
      out_type=jax.ShapeDtypeStruct((_HIST, 8, _NW, 8, 128), jnp.float32),
      mesh=mesh,
      scratch_types=[
          pltpu.VMEM((_HIST, _BT), jnp.int32),
          pltpu.VMEM((_BT, _EMBED_DIM), jnp.float32),
          pltpu.VMEM((_BT, _EMBED_DIM), jnp.float32),
          pltpu.VMEM((8, 8, 128), jnp.float32),
          pltpu.VMEM((8, 8, 128), jnp.float32),
          pltpu.SemaphoreType.DMA,
          pltpu.SemaphoreType.DMA,
          pltpu.SemaphoreType.DMA,
          pltpu.SemaphoreType.DMA,
      ],
      compiler_params=pltpu.CompilerParams(use_tc_tiling_on_sc=False,
                                           needs_layout_passes=False),
  )(table, idx_t)
  return out5.transpose(2, 4, 0, 1, 3).reshape(_BATCH, _HIST, _EMBED_DIM)


# speedup vs baseline: 1.1828x; 1.1828x over previous
import jax
import jax.numpy as jnp
from jax import lax
from jax.experimental import pallas as pl
from jax.experimental.pallas import tpu as pltpu
from jax.experimental.pallas import tpu_sc as plsc

_VOCAB = 100000
_EMBED_DIM = 64
_BATCH = 4096
_HIST = 50

_NC = 2
_NS = 16
_NW = _NC * _NS

_BT = _BATCH // _NW
_L = 16


def _transpose_block(buf, buf_t):
  iota = lax.iota(jnp.int32, _L)
  row_vecs = [v * _L + iota for v in range(_BT // _L)]

  def te_body(te, carry):
    base = te * 8
    for se in range(8):
      col = jnp.full((_L,), base + se, jnp.int32)
      vals = [plsc.load_gather(buf, [rows, col]) for rows in row_vecs]
      for v in range(_BT // _L):
        buf_t[te, se, pl.ds(v * _L, _L)] = vals[v]
    return carry

  lax.fori_loop(0, 8, te_body, 0)


def _gather_body(table_hbm, idx_hbm, out_hbm, idx_v, buf0, buf1, bt0, bt1,
                 sem_g0, sem_g1, sem_w0, sem_w1):
  wid = lax.axis_index("s") * _NC + lax.axis_index("c")
  pltpu.sync_copy(idx_hbm.at[pl.ds(0, _HIST), pl.ds(wid * _BT, _BT)], idx_v)

  pltpu.async_copy(table_hbm.at[idx_v.at[0]], buf0, sem_g0)

  def step(h, buf, buf_t, sem_g, sem_w, first, last):
    pltpu.make_async_copy(table_hbm.at[idx_v.at[h]], buf, sem_g).wait()

    @pl.when(jnp.logical_not(first))
    def _drain_prev_write():
      pltpu.make_async_copy(buf_t, out_hbm.at[h - 2, pl.ds(0, 8), wid],
                            sem_w).wait()

    _transpose_block(buf, buf_t)
    pltpu.async_copy(buf_t, out_hbm.at[h, pl.ds(0, 8), wid], sem_w)

  def superstep(i, carry):
    h0 = 2 * i
    h1 = 2 * i + 1
    pltpu.async_copy(table_hbm.at[idx_v.at[h1]], buf1, sem_g1)
    step(h0, buf0, bt0, sem_g0, sem_w0, i == 0, False)

    @pl.when(i < _HIST // 2 - 1)
    def _fire_next():
      pltpu.async_copy(table_hbm.at[idx_v.at[h1 + 1]], buf0, sem_g0)

    step(h1, buf1, bt1, sem_g1, sem_w1, i == 0, i == _HIST // 2 - 1)
    return carry

  lax.fori_loop(0, _HIST // 2, superstep, 0)
  pltpu.make_async_copy(bt0, out_hbm.at[_HIST - 2, pl.ds(0, 8), wid],
                        sem_w0).wait()
  pltpu.make_async_copy(bt1, out_hbm.at[_HIST - 1, pl.ds(0, 8), wid],
                        sem_w1).wait()


@jax.jit
def kernel(indices, table):
  idx_t = indices.T.astype(jnp.int32)
  mesh = plsc.VectorSubcoreMesh(core_axis_name="c", subcore_axis_name="s")
  out5 = pl.kernel(
      _gather_body,
      out_type=jax.ShapeDtypeStruct((_HIST, 8, _NW, 8, 128), jnp.float32),
      mesh=mesh,
      scratch_types=[
          pltpu.VMEM((_HIST, _BT), jnp.int32),
          pltpu.VMEM((_BT, _EMBED_DIM), jnp.float32),
          pltpu.VMEM((_BT, _EMBED_DIM), jnp.float32),
          pltpu.VMEM((8, 8, 128), jnp.float32),
          pltpu.VMEM((8, 8, 128), jnp.float32),
          pltpu.SemaphoreType.DMA,
          pltpu.SemaphoreType.DMA,
          pltpu.SemaphoreType.DMA,
          pltpu.SemaphoreType.DMA,
      ],
      compiler_params=pltpu.CompilerParams(use_tc_tiling_on_sc=False,
                                           needs_layout_passes=False),
  )(table, idx_t)
  return out5.transpose(2, 4, 0, 1, 3).reshape(_BATCH, _HIST, _EMBED_DIM)
